# baseline (speedup 1.0000x reference)
import functools

import jax
import jax.numpy as jnp
from jax import lax
from jax.experimental import pallas as pl
from jax.experimental.pallas import tpu as pltpu
from jax.experimental.pallas import tpu_sc as plsc

NW = 32
L = 16
KG = 16
GN = 16
ESTAGE = 8192


@functools.cache
def _make_edge_kernel(n_nodes, O):
    C8 = O // L // 4
    n_out = n_nodes + 1
    mesh = plsc.VectorSubcoreMesh(
        core_axis_name="c", subcore_axis_name="s", num_cores=2,
        num_subcores=16)

    def body(xl, xr, srcp, dstp, info, att, bias, out,
             src_v, dst_v, rows0, rows1, xrs_v, acc_v, den_b, att_v, bias_v,
             outs_v, info_v, sem0, sem1):
        wid = lax.axis_index("s") * 2 + lax.axis_index("c")
        pltpu.sync_copy(info.at[wid], info_v)
        iv = info_v[...]
        a0 = pl.multiple_of(iv[0], 32)
        nblk2 = iv[1]
        lo_node = iv[2]
        hi_node = iv[3]
        pltpu.sync_copy(srcp.at[pl.ds(a0, ESTAGE)], src_v)
        pltpu.sync_copy(dstp.at[pl.ds(a0, ESTAGE)], dst_v)
        pltpu.sync_copy(att, att_v)
        pltpu.sync_copy(bias, bias_v)

        zero16 = jnp.zeros((L,), jnp.float32)

        def zacc(i, _):
            def zc(c8, _):
                for u in range(4):
                    acc_v[i, pl.ds((c8 * 4 + u) * L, L)] = zero16
                return 0

            lax.fori_loop(0, C8, zc, 0)
            den_b[i, :] = zero16
            return 0

        lax.fori_loop(0, 2 * GN, zacc, 0)

        def transform(gbase, i0, i1):
            roff = lax.rem(gbase, 2 * GN)

            def tr(i, _):
                invv = 1.0 / (den_b[roff + i, :] + jnp.float32(1e-16))

                @plsc.parallel_loop(0, C8, unroll=2)
                def _(c8):
                    for u in range(4):
                        sl = pl.ds((c8 * 4 + u) * L, L)
                        outs_v[i, sl] = jnp.maximum(
                            acc_v[roff + i, sl] * invv + bias_v[sl], 0.0)
                return 0

            lax.fori_loop(i0, i1, tr, 0)

        def flush_group(pbase):
            b = pl.multiple_of(pbase, GN)
            transform(b, jnp.maximum(lo_node - b, 0), GN)

            @pl.when(b >= lo_node)
            def _():
                pltpu.sync_copy(outs_v, out.at[pl.ds(b, GN)])

            @pl.when(b < lo_node)
            def _():
                def pf(i, _):
                    pltpu.sync_copy(outs_v.at[i], out.at[b + i])
                    return 0

                lax.fori_loop(lo_node - b, GN, pf, 0)

        last_off = jnp.maximum(nblk2 * 2 - 1, 0) * KG

        def process(rows, sem, blk, nxt, carry):
            pltpu.make_async_copy(
                xl.at[src_v.at[pl.ds(0, KG)]], rows, sem).wait()
            m, den, cur, xslot, aslot, pbase = carry
            base = pl.multiple_of(blk * KG, KG)
            dv = dst_v[pl.ds(base, KG)]
            es = []
            chs = []
            prev_curs = []
            prev_aslots = []
            aslots = []
            owns = []
            for j in range(KG):
                dnew = dv[j]
                own = jnp.logical_and(dnew >= lo_node, dnew < hi_node)
                change = jnp.logical_and(own, dnew != cur)
                nslot = lax.rem(dnew, GN)
                need_load = jnp.logical_and(
                    change, jnp.logical_or(nslot == 0, cur == n_nodes))
                prev_curs.append(cur)
                prev_aslots.append(aslot)
                cur = jnp.where(change, dnew, cur)
                xslot = jnp.where(change, nslot, xslot)
                aslot = jnp.where(change, lax.rem(dnew, 2 * GN), aslot)
                aslots.append(aslot)

                @pl.when(need_load)
                def _():
                    xb = pl.multiple_of(dnew - nslot, GN)
                    pltpu.sync_copy(xr.at[pl.ds(xb, GN)], xrs_v)

                @plsc.parallel_loop(0, C8, unroll=2, carry=zero16)
                def sacc(c8, s):
                    for u in range(4):
                        sl = pl.ds((c8 * 4 + u) * L, L)
                        mv = rows[j, sl] + xrs_v[xslot, sl]
                        lr = jnp.where(mv > 0, mv, jnp.float32(0.2) * mv)
                        s = s + att_v[sl] * lr
                    return s
                es.append(jnp.where(own, jnp.sum(sacc), jnp.float32(-3e38)))
                chs.append(change)
                owns.append(own)

            for j in range(KG):
                change = chs[j]

                @pl.when(change)
                def _():
                    den_b[prev_aslots[j], :] = den

                completed = jnp.logical_and(
                    jnp.logical_and(change, prev_curs[j] < hi_node),
                    lax.rem(prev_curs[j], GN) == GN - 1)
                pbase = jnp.where(completed, prev_curs[j] - (GN - 1), pbase)
                m = jnp.where(change, jnp.float32(-3e38), m)
                den = jnp.where(change, jnp.zeros_like(den), den)
                d = es[j] - m
                pos = d >= 0
                z_v = jnp.exp(jnp.full((L,), -jnp.abs(d), jnp.float32))
                scale_v = jnp.where(pos, z_v, jnp.float32(1.0))
                w_v = jnp.where(jnp.logical_and(owns[j], pos),
                                jnp.float32(1.0),
                                jnp.where(owns[j], z_v, jnp.float32(0.0)))
                den = den * scale_v + w_v
                m = jnp.where(pos, es[j], m)

                @plsc.parallel_loop(0, C8, unroll=2)
                def _(c8):
                    for u in range(4):
                        sl = pl.ds((c8 * 4 + u) * L, L)
                        acc_v[aslots[j], sl] = (acc_v[aslots[j], sl]
                                                * scale_v
                                                + w_v * rows[j, sl])

            @pl.when(pbase >= 0)
            def _():
                flush_group(pbase)

            pbase = jnp.int32(-1)
            off = pl.multiple_of(jnp.minimum(nxt * KG, last_off), KG)
            pltpu.async_copy(xl.at[src_v.at[pl.ds(off, KG)]], rows, sem)
            return (m, den, cur, xslot, aslot, pbase)

        @pl.when(nblk2 > 0)
        def _():
            pltpu.async_copy(xl.at[src_v.at[pl.ds(0, KG)]], rows0, sem0)
            pltpu.async_copy(xl.at[src_v.at[pl.ds(KG, KG)]], rows1, sem1)
            carry0 = (jnp.float32(-3e38), jnp.zeros((L,), jnp.float32),
                      jnp.int32(n_nodes), jnp.int32(0), jnp.int32(0),
                      jnp.int32(-1))

            def outer(i, carry):
                carry = process(rows0, sem0, 2 * i, 2 * i + 2, carry)
                carry = process(rows1, sem1, 2 * i + 1, 2 * i + 3, carry)
                return carry

            m, den, cur, xslot, aslot, pbase = lax.fori_loop(
                0, nblk2, outer, carry0)
            pltpu.make_async_copy(
                xl.at[src_v.at[pl.ds(0, KG)]], rows0, sem0).wait()
            pltpu.make_async_copy(
                xl.at[src_v.at[pl.ds(0, KG)]], rows1, sem1).wait()

            @pl.when(cur < hi_node)
            def _():
                den_b[aslot, :] = den
                slot = lax.rem(cur, GN)
                gbase = cur - slot
                s0 = jnp.maximum(lo_node - gbase, 0)
                transform(gbase, s0, slot + 1)

                def tf(i, _):
                    pltpu.sync_copy(outs_v.at[i], out.at[gbase + i])
                    return 0

                lax.fori_loop(s0, slot + 1, tf, 0)

    return pl.kernel(
        body,
        out_type=jax.ShapeDtypeStruct((n_out, O), jnp.float32),
        mesh=mesh,
        compiler_params=pltpu.CompilerParams(needs_layout_passes=False),
        scratch_types=[
            pltpu.VMEM((ESTAGE,), jnp.int32),
            pltpu.VMEM((ESTAGE,), jnp.int32),
            pltpu.VMEM((KG, O), jnp.float32),
            pltpu.VMEM((KG, O), jnp.float32),
            pltpu.VMEM((GN, O), jnp.float32),
            pltpu.VMEM((2 * GN, O), jnp.float32),
            pltpu.VMEM((2 * GN, L), jnp.float32),
            pltpu.VMEM((O,), jnp.float32),
            pltpu.VMEM((O,), jnp.float32),
            pltpu.VMEM((GN, O), jnp.float32),
            pltpu.VMEM((16,), jnp.int32),
            pltpu.SemaphoreType.DMA,
            pltpu.SemaphoreType.DMA,
        ],
    )


def _proj_body(x_ref, wl_ref, wr_ref, xl_ref, xr_ref):
    xl_ref[...] = jnp.dot(x_ref[...], wl_ref[...],
                          preferred_element_type=jnp.float32)
    xr_ref[...] = jnp.dot(x_ref[...], wr_ref[...],
                          preferred_element_type=jnp.float32)


def _proj(x, wl, wr, block_rows=1000):
    n, k = x.shape
    o = wl.shape[1]
    n_pad = n + GN
    grid = (pl.cdiv(n_pad, block_rows),)
    return pl.pallas_call(
        _proj_body,
        grid=grid,
        in_specs=[
            pl.BlockSpec((block_rows, k), lambda i: (i, 0)),
            pl.BlockSpec((k, o), lambda i: (0, 0)),
            pl.BlockSpec((k, o), lambda i: (0, 0)),
        ],
        out_specs=[
            pl.BlockSpec((block_rows, o), lambda i: (i, 0)),
            pl.BlockSpec((block_rows, o), lambda i: (i, 0)),
        ],
        out_shape=[
            jax.ShapeDtypeStruct((n_pad, o), jnp.float32),
            jax.ShapeDtypeStruct((n_pad, o), jnp.float32),
        ],
    )(x, wl, wr)


def _gather_body(last_ref, h_ref, o_ref):
    o_ref[...] = h_ref[...]


def _readout(h, last):
    b = last.shape[0]
    n, f = h.shape
    grid_spec = pltpu.PrefetchScalarGridSpec(
        num_scalar_prefetch=1,
        grid=(b,),
        in_specs=[pl.BlockSpec((1, 1, f),
                               lambda i, last_ref: (last_ref[i], 0, 0))],
        out_specs=pl.BlockSpec((1, 1, f), lambda i, last_ref: (i, 0, 0)),
    )
    out = pl.pallas_call(
        _gather_body,
        grid_spec=grid_spec,
        out_shape=jax.ShapeDtypeStruct((b, 1, f), jnp.float32),
    )(last, h.reshape(n, 1, f))
    return out.reshape(b, f)


def _head_body(z_ref, m1_ref, b1_ref, m2_ref, b2_ref, m3_ref, c3_ref, o_ref):
    z = z_ref[...]
    z = jnp.maximum(jnp.dot(z, m1_ref[...],
                            preferred_element_type=jnp.float32) + b1_ref[...],
                    0.0)
    z = jnp.maximum(jnp.dot(z, m2_ref[...],
                            preferred_element_type=jnp.float32) + b2_ref[...],
                    0.0)
    o_ref[...] = jnp.dot(z, m3_ref[...],
                         preferred_element_type=jnp.float32) + c3_ref[...]


def _head(z, m1, b1, m2, b2, m3, c3):
    return pl.pallas_call(
        _head_body,
        out_shape=jax.ShapeDtypeStruct((z.shape[0], 1), jnp.float32),
    )(z, m1, b1[None, :], m2, b2[None, :], m3, c3[None, :])


def _prep_edges(edge_index, n):
    e = edge_index.shape[1]
    e2 = e + n
    shift = max(int(e2 - 1).bit_length(), 1)
    loops = jnp.arange(n, dtype=jnp.int32)
    src = jnp.concatenate([edge_index[0], loops])
    dst = jnp.concatenate([edge_index[1], loops])
    key = (dst.astype(jnp.uint32) << shift) | jnp.arange(
        e2, dtype=jnp.uint32)
    ks = jnp.sort(key)
    dst_s = (ks >> shift).astype(jnp.int32)
    order = (ks & jnp.uint32((1 << shift) - 1)).astype(jnp.int32)
    src_s = src[order]
    src_s = jnp.concatenate([src_s, jnp.zeros((ESTAGE,), jnp.int32)])
    dst_s = jnp.concatenate([dst_s, jnp.full((ESTAGE,), n, jnp.int32)])
    rp = jnp.searchsorted(dst_s, jnp.arange(n + 1, dtype=jnp.int32))
    rp = rp.astype(jnp.int32)
    targets = (jnp.arange(NW + 1, dtype=jnp.int32) * e2) // NW
    nb = jnp.searchsorted(rp, targets, side="left").astype(jnp.int32)
    e_lo = rp[nb]
    eb = (e_lo[:NW] // 32) * 32
    nblk2 = jnp.minimum((e_lo[1:] - eb + 31) // 32, ESTAGE // 32)
    info = jnp.zeros((NW, 16), jnp.int32)
    info = info.at[:, 0].set(eb)
    info = info.at[:, 1].set(nblk2)
    info = info.at[:, 2].set(nb[:NW])
    info = info.at[:, 3].set(nb[1:])
    return src_s, dst_s, info


def _gat_layer(h, src_pad, dst_pad, info, wl, wr, att, b, n):
    xl, xr = _proj(h, wl, wr)
    edge_k = _make_edge_kernel(n, wl.shape[1])
    out = edge_k(xl, xr, src_pad, dst_pad, info, att, b)
    return out[:n]


def kernel(x, edge_index, batch, cond, W1l, W1r, att1, b1, W2l, W2r, att2, b2,
           W3l, W3r, att3, b3, W4l, W4r, att4, b4, M1, c1, g1, be1, M2, c2,
           g2, be2, M3, c3):
    n = x.shape[0]
    nb_graphs = cond.shape[0]
    src_pad, dst_pad, info = _prep_edges(edge_index, n)
    h = _gat_layer(x, src_pad, dst_pad, info, W1l, W1r, att1, b1, n)
    h = _gat_layer(h, src_pad, dst_pad, info, W2l, W2r, att2, b2, n)
    h = _gat_layer(h, src_pad, dst_pad, info, W3l, W3r, att3, b3, n)
    h = _gat_layer(h, src_pad, dst_pad, info, W4l, W4r, att4, b4, n)

    last = jnp.searchsorted(batch, jnp.arange(nb_graphs, dtype=jnp.int32),
                            side="right").astype(jnp.int32) - 1
    last = jnp.clip(last, 0, n - 1)
    g = _readout(h, last)
    z = jnp.concatenate([g, cond], axis=1)

    inv = 1.0 / jnp.sqrt(1.0 + 1e-5)
    s1 = g1 * inv
    m1 = M1 * s1[None, :]
    bb1 = c1 * s1 + be1
    s2 = g2 * inv
    m2 = M2 * s2[None, :]
    bb2 = c2 * s2 + be2
    return _head(z, m1, bb1, m2, bb2, M3, c3)


# speedup vs baseline: 1.0315x; 1.0315x over previous
import functools

import jax
import jax.numpy as jnp
from jax import lax
from jax.experimental import pallas as pl
from jax.experimental.pallas import tpu as pltpu
from jax.experimental.pallas import tpu_sc as plsc

NW = 32
L = 16
KG = 16
GN = 16
ESTAGE = 8192


@functools.cache
def _make_edge_kernel(n_nodes, O):
    C8 = O // L // 4
    n_out = n_nodes + 1
    mesh = plsc.VectorSubcoreMesh(
        core_axis_name="c", subcore_axis_name="s", num_cores=2,
        num_subcores=16)

    def body(xl, xr, srcp, dstp, info, att, bias, out,
             src_v, dst_v, rows0, rows1, xrs_v, acc_v, den_b, att_v, bias_v,
             outs_v, info_v, sem0, sem1):
        wid = lax.axis_index("s") * 2 + lax.axis_index("c")
        pltpu.sync_copy(info.at[wid], info_v)
        iv = info_v[...]
        a0 = pl.multiple_of(iv[0], 32)
        nblk2 = iv[1]
        lo_node = iv[2]
        hi_node = iv[3]
        pltpu.sync_copy(srcp.at[pl.ds(a0, ESTAGE)], src_v)
        pltpu.sync_copy(dstp.at[pl.ds(a0, ESTAGE)], dst_v)
        pltpu.sync_copy(att, att_v)
        pltpu.sync_copy(bias, bias_v)

        zero16 = jnp.zeros((L,), jnp.float32)

        def zacc(i, _):
            def zc(c8, _):
                for u in range(4):
                    acc_v[i, pl.ds((c8 * 4 + u) * L, L)] = zero16
                return 0

            lax.fori_loop(0, C8, zc, 0)
            den_b[i, :] = zero16
            return 0

        lax.fori_loop(0, 2 * GN, zacc, 0)

        def transform(gbase, i0, i1):
            roff = lax.rem(gbase, 2 * GN)

            def tr(i, _):
                invv = 1.0 / (den_b[roff + i, :] + jnp.float32(1e-16))

                @plsc.parallel_loop(0, C8)
                def _(c8):
                    for u in range(4):
                        sl = pl.ds((c8 * 4 + u) * L, L)
                        outs_v[i, sl] = jnp.maximum(
                            acc_v[roff + i, sl] * invv + bias_v[sl], 0.0)
                return 0

            lax.fori_loop(i0, i1, tr, 0)

        def flush_group(pbase):
            b = pl.multiple_of(pbase, GN)
            transform(b, jnp.maximum(lo_node - b, 0), GN)

            @pl.when(b >= lo_node)
            def _():
                pltpu.sync_copy(outs_v, out.at[pl.ds(b, GN)])

            @pl.when(b < lo_node)
            def _():
                def pf(i, _):
                    pltpu.sync_copy(outs_v.at[i], out.at[b + i])
                    return 0

                lax.fori_loop(lo_node - b, GN, pf, 0)

        last_off = jnp.maximum(nblk2 * 2 - 1, 0) * KG

        def process(rows, sem, blk, nxt, carry):
            pltpu.make_async_copy(
                xl.at[src_v.at[pl.ds(0, KG)]], rows, sem).wait()
            m, den, cur, xslot, aslot, pbase = carry
            base = pl.multiple_of(blk * KG, KG)
            dv = dst_v[pl.ds(base, KG)]
            es = []
            chs = []
            prev_curs = []
            prev_aslots = []
            aslots = []
            owns = []
            for j in range(KG):
                dnew = dv[j]
                own = jnp.logical_and(dnew >= lo_node, dnew < hi_node)
                change = jnp.logical_and(own, dnew != cur)
                nslot = lax.rem(dnew, GN)
                need_load = jnp.logical_and(
                    change, jnp.logical_or(nslot == 0, cur == n_nodes))
                prev_curs.append(cur)
                prev_aslots.append(aslot)
                cur = jnp.where(change, dnew, cur)
                xslot = jnp.where(change, nslot, xslot)
                aslot = jnp.where(change, lax.rem(dnew, 2 * GN), aslot)
                aslots.append(aslot)

                @pl.when(need_load)
                def _():
                    xb = pl.multiple_of(dnew - nslot, GN)
                    pltpu.sync_copy(xr.at[pl.ds(xb, GN)], xrs_v)

                @plsc.parallel_loop(0, C8, carry=zero16)
                def sacc(c8, s):
                    for u in range(4):
                        sl = pl.ds((c8 * 4 + u) * L, L)
                        mv = rows[j, sl] + xrs_v[xslot, sl]
                        lr = jnp.where(mv > 0, mv, jnp.float32(0.2) * mv)
                        s = s + att_v[sl] * lr
                    return s
                es.append(jnp.where(own, jnp.sum(sacc), jnp.float32(-3e38)))
                chs.append(change)
                owns.append(own)

            for j in range(KG):
                change = chs[j]

                @pl.when(change)
                def _():
                    den_b[prev_aslots[j], :] = den

                completed = jnp.logical_and(
                    jnp.logical_and(change, prev_curs[j] < hi_node),
                    lax.rem(prev_curs[j], GN) == GN - 1)
                pbase = jnp.where(completed, prev_curs[j] - (GN - 1), pbase)
                m = jnp.where(change, jnp.float32(-3e38), m)
                den = jnp.where(change, jnp.zeros_like(den), den)
                d = es[j] - m
                pos = d >= 0
                z_v = jnp.exp(jnp.full((L,), -jnp.abs(d), jnp.float32))
                scale_v = jnp.where(pos, z_v, jnp.float32(1.0))
                w_v = jnp.where(jnp.logical_and(owns[j], pos),
                                jnp.float32(1.0),
                                jnp.where(owns[j], z_v, jnp.float32(0.0)))
                den = den * scale_v + w_v
                m = jnp.where(pos, es[j], m)

                @plsc.parallel_loop(0, C8)
                def _(c8):
                    for u in range(4):
                        sl = pl.ds((c8 * 4 + u) * L, L)
                        acc_v[aslots[j], sl] = (acc_v[aslots[j], sl]
                                                * scale_v
                                                + w_v * rows[j, sl])

            @pl.when(pbase >= 0)
            def _():
                flush_group(pbase)

            pbase = jnp.int32(-1)
            off = pl.multiple_of(jnp.minimum(nxt * KG, last_off), KG)
            pltpu.async_copy(xl.at[src_v.at[pl.ds(off, KG)]], rows, sem)
            return (m, den, cur, xslot, aslot, pbase)

        @pl.when(nblk2 > 0)
        def _():
            pltpu.async_copy(xl.at[src_v.at[pl.ds(0, KG)]], rows0, sem0)
            pltpu.async_copy(xl.at[src_v.at[pl.ds(KG, KG)]], rows1, sem1)
            carry0 = (jnp.float32(-3e38), jnp.zeros((L,), jnp.float32),
                      jnp.int32(n_nodes), jnp.int32(0), jnp.int32(0),
                      jnp.int32(-1))

            def outer(i, carry):
                carry = process(rows0, sem0, 2 * i, 2 * i + 2, carry)
                carry = process(rows1, sem1, 2 * i + 1, 2 * i + 3, carry)
                return carry

            m, den, cur, xslot, aslot, pbase = lax.fori_loop(
                0, nblk2, outer, carry0)
            pltpu.make_async_copy(
                xl.at[src_v.at[pl.ds(0, KG)]], rows0, sem0).wait()
            pltpu.make_async_copy(
                xl.at[src_v.at[pl.ds(0, KG)]], rows1, sem1).wait()

            @pl.when(cur < hi_node)
            def _():
                den_b[aslot, :] = den
                slot = lax.rem(cur, GN)
                gbase = cur - slot
                s0 = jnp.maximum(lo_node - gbase, 0)
                transform(gbase, s0, slot + 1)

                def tf(i, _):
                    pltpu.sync_copy(outs_v.at[i], out.at[gbase + i])
                    return 0

                lax.fori_loop(s0, slot + 1, tf, 0)

    return pl.kernel(
        body,
        out_type=jax.ShapeDtypeStruct((n_out, O), jnp.float32),
        mesh=mesh,
        compiler_params=pltpu.CompilerParams(needs_layout_passes=False),
        scratch_types=[
            pltpu.VMEM((ESTAGE,), jnp.int32),
            pltpu.VMEM((ESTAGE,), jnp.int32),
            pltpu.VMEM((KG, O), jnp.float32),
            pltpu.VMEM((KG, O), jnp.float32),
            pltpu.VMEM((GN, O), jnp.float32),
            pltpu.VMEM((2 * GN, O), jnp.float32),
            pltpu.VMEM((2 * GN, L), jnp.float32),
            pltpu.VMEM((O,), jnp.float32),
            pltpu.VMEM((O,), jnp.float32),
            pltpu.VMEM((GN, O), jnp.float32),
            pltpu.VMEM((16,), jnp.int32),
            pltpu.SemaphoreType.DMA,
            pltpu.SemaphoreType.DMA,
        ],
    )


def _proj_body(x_ref, wl_ref, wr_ref, xl_ref, xr_ref):
    xl_ref[...] = jnp.dot(x_ref[...], wl_ref[...],
                          preferred_element_type=jnp.float32)
    xr_ref[...] = jnp.dot(x_ref[...], wr_ref[...],
                          preferred_element_type=jnp.float32)


def _proj(x, wl, wr, block_rows=1000):
    n, k = x.shape
    o = wl.shape[1]
    n_pad = n + GN
    grid = (pl.cdiv(n_pad, block_rows),)
    return pl.pallas_call(
        _proj_body,
        grid=grid,
        in_specs=[
            pl.BlockSpec((block_rows, k), lambda i: (i, 0)),
            pl.BlockSpec((k, o), lambda i: (0, 0)),
            pl.BlockSpec((k, o), lambda i: (0, 0)),
        ],
        out_specs=[
            pl.BlockSpec((block_rows, o), lambda i: (i, 0)),
            pl.BlockSpec((block_rows, o), lambda i: (i, 0)),
        ],
        out_shape=[
            jax.ShapeDtypeStruct((n_pad, o), jnp.float32),
            jax.ShapeDtypeStruct((n_pad, o), jnp.float32),
        ],
    )(x, wl, wr)


def _gather_body(last_ref, h_ref, o_ref):
    o_ref[...] = h_ref[...]


def _readout(h, last):
    b = last.shape[0]
    n, f = h.shape
    grid_spec = pltpu.PrefetchScalarGridSpec(
        num_scalar_prefetch=1,
        grid=(b,),
        in_specs=[pl.BlockSpec((1, 1, f),
                               lambda i, last_ref: (last_ref[i], 0, 0))],
        out_specs=pl.BlockSpec((1, 1, f), lambda i, last_ref: (i, 0, 0)),
    )
    out = pl.pallas_call(
        _gather_body,
        grid_spec=grid_spec,
        out_shape=jax.ShapeDtypeStruct((b, 1, f), jnp.float32),
    )(last, h.reshape(n, 1, f))
    return out.reshape(b, f)


def _head_body(z_ref, m1_ref, b1_ref, m2_ref, b2_ref, m3_ref, c3_ref, o_ref):
    z = z_ref[...]
    z = jnp.maximum(jnp.dot(z, m1_ref[...],
                            preferred_element_type=jnp.float32) + b1_ref[...],
                    0.0)
    z = jnp.maximum(jnp.dot(z, m2_ref[...],
                            preferred_element_type=jnp.float32) + b2_ref[...],
                    0.0)
    o_ref[...] = jnp.dot(z, m3_ref[...],
                         preferred_element_type=jnp.float32) + c3_ref[...]


def _head(z, m1, b1, m2, b2, m3, c3):
    return pl.pallas_call(
        _head_body,
        out_shape=jax.ShapeDtypeStruct((z.shape[0], 1), jnp.float32),
    )(z, m1, b1[None, :], m2, b2[None, :], m3, c3[None, :])


def _prep_edges(edge_index, n):
    e = edge_index.shape[1]
    e2 = e + n
    shift = max(int(e2 - 1).bit_length(), 1)
    loops = jnp.arange(n, dtype=jnp.int32)
    src = jnp.concatenate([edge_index[0], loops])
    dst = jnp.concatenate([edge_index[1], loops])
    key = (dst.astype(jnp.uint32) << shift) | jnp.arange(
        e2, dtype=jnp.uint32)
    ks = jnp.sort(key)
    dst_s = (ks >> shift).astype(jnp.int32)
    order = (ks & jnp.uint32((1 << shift) - 1)).astype(jnp.int32)
    src_s = src[order]
    src_s = jnp.concatenate([src_s, jnp.zeros((ESTAGE,), jnp.int32)])
    dst_s = jnp.concatenate([dst_s, jnp.full((ESTAGE,), n, jnp.int32)])
    rp = jnp.searchsorted(dst_s, jnp.arange(n + 1, dtype=jnp.int32))
    rp = rp.astype(jnp.int32)
    targets = (jnp.arange(NW + 1, dtype=jnp.int32) * e2) // NW
    nb = jnp.searchsorted(rp, targets, side="left").astype(jnp.int32)
    e_lo = rp[nb]
    eb = (e_lo[:NW] // 32) * 32
    nblk2 = jnp.minimum((e_lo[1:] - eb + 31) // 32, ESTAGE // 32)
    info = jnp.zeros((NW, 16), jnp.int32)
    info = info.at[:, 0].set(eb)
    info = info.at[:, 1].set(nblk2)
    info = info.at[:, 2].set(nb[:NW])
    info = info.at[:, 3].set(nb[1:])
    return src_s, dst_s, info


def _gat_layer(h, src_pad, dst_pad, info, wl, wr, att, b, n):
    xl, xr = _proj(h, wl, wr)
    edge_k = _make_edge_kernel(n, wl.shape[1])
    out = edge_k(xl, xr, src_pad, dst_pad, info, att, b)
    return out[:n]


def kernel(x, edge_index, batch, cond, W1l, W1r, att1, b1, W2l, W2r, att2, b2,
           W3l, W3r, att3, b3, W4l, W4r, att4, b4, M1, c1, g1, be1, M2, c2,
           g2, be2, M3, c3):
    n = x.shape[0]
    nb_graphs = cond.shape[0]
    src_pad, dst_pad, info = _prep_edges(edge_index, n)
    h = _gat_layer(x, src_pad, dst_pad, info, W1l, W1r, att1, b1, n)
    h = _gat_layer(h, src_pad, dst_pad, info, W2l, W2r, att2, b2, n)
    h = _gat_layer(h, src_pad, dst_pad, info, W3l, W3r, att3, b3, n)
    h = _gat_layer(h, src_pad, dst_pad, info, W4l, W4r, att4, b4, n)

    last = jnp.searchsorted(batch, jnp.arange(nb_graphs, dtype=jnp.int32),
                            side="right").astype(jnp.int32) - 1
    last = jnp.clip(last, 0, n - 1)
    g = _readout(h, last)
    z = jnp.concatenate([g, cond], axis=1)

    inv = 1.0 / jnp.sqrt(1.0 + 1e-5)
    s1 = g1 * inv
    m1 = M1 * s1[None, :]
    bb1 = c1 * s1 + be1
    s2 = g2 * inv
    m2 = M2 * s2[None, :]
    bb2 = c2 * s2 + be2
    return _head(z, m1, bb1, m2, bb2, M3, c3)
